# baseline (speedup 1.0000x reference)
import functools

import jax
import jax.numpy as jnp
from jax import lax
from jax.experimental import pallas as pl
from jax.experimental.pallas import tpu as pltpu
from jax.experimental.pallas import tpu_sc as plsc

N = 10000
E = 320000
F = 128
NG = 16
N_P = 10112
PAD = N_P - N
DUMMY = N
NW = 32
CH = 80
EW = CH * 128
EP = EW * NW
EPR = EP // 128
BR = 632
NBLK = N_P // BR
JR = N_P // 128
TW_BIG = 128
TW_SMALL = 16
NHALF = N_P // 2
ACC_ROWS = 5120
DUMMY_L = NHALF


def _seg_body(TW, table_h, keep_h, a_h, b_h, out_h, keep_v, ab, bb,
              r0, r1, r2, r3, g0, g1, g2, g3, s0, s1, s2, s3, acc):
    cid = lax.axis_index("c")
    sid = lax.axis_index("s")
    lo = cid * NHALF
    bufs = (r0, r1, r2, r3)
    gsem = (g0, g1, g2, g3)
    ssem = (s0, s1, s2, s3)

    pltpu.sync_copy(keep_h, keep_v)

    @pl.loop(0, 128)
    def _z(r):
        for kk in range(TW // 16):
            r0[r, pl.ds(kk * 16, 16)] = jnp.zeros((16,), jnp.float32)

    zbase = sid * (ACC_ROWS // 16)
    for q in range(2):
        pltpu.sync_copy(r0, acc.at[pl.ds(zbase + q * 128, 128)])
    pltpu.sync_copy(r0.at[pl.ds(0, 64)], acc.at[pl.ds(zbase + 256, 64)])
    plsc.subcore_barrier()

    row0 = sid * (EPR // 16)

    @pl.loop(0, EPR // 256)
    def _m(m):
        mrow = row0 + m * 16
        pltpu.sync_copy(a_h.at[pl.ds(mrow, 16)], ab)
        pltpu.sync_copy(b_h.at[pl.ds(mrow, 16)], bb)

        @pl.loop(0, 16)
        def _c(j):
            @pl.loop(0, 8)
            def _k(k):
                a16 = ab[j, pl.ds(k * 16, 16)]
                b16 = bb[j, pl.ds(k * 16, 16)]
                em = (plsc.load_gather(keep_v, [a16])
                      * plsc.load_gather(keep_v, [b16]))
                dl = b16 - lo
                live = (em > 0.0) & (dl >= 0) & (dl < NHALF)
                bb[j, pl.ds(k * 16, 16)] = jnp.where(
                    live, dl, jnp.full((16,), DUMMY_L, jnp.int32))

        for q in range(3):
            pltpu.async_copy(table_h.at[ab.at[q]], bufs[q], gsem[q])
        for q in range(16):
            b = q % 4
            pltpu.make_async_copy(table_h.at[ab.at[q]], bufs[b],
                                  gsem[b]).wait()
            pltpu.async_copy(bufs[b], acc.at[bb.at[q]], ssem[b], add=True)
            if q + 3 < 16:
                nb = (q + 3) % 4
                if q >= 1:
                    pltpu.make_async_copy(bufs[nb], acc.at[bb.at[q - 1]],
                                          ssem[nb]).wait()
                pltpu.async_copy(table_h.at[ab.at[q + 3]], bufs[nb],
                                 gsem[nb])
        for q in range(12, 16):
            pltpu.make_async_copy(bufs[q % 4], acc.at[bb.at[q]],
                                  ssem[q % 4]).wait()

    plsc.subcore_barrier()

    @pl.when(sid < 8)
    def _out():
        obase = sid * BR
        for q in range(4):
            pltpu.sync_copy(acc.at[pl.ds(obase + q * 128, 128)],
                            out_h.at[pl.ds(lo + obase + q * 128, 128)])
        pltpu.sync_copy(acc.at[pl.ds(obase + 512, 120)],
                        out_h.at[pl.ds(lo + obase + 512, 120)])


def _make_seg(TW):
    mesh = plsc.VectorSubcoreMesh(core_axis_name="c", subcore_axis_name="s")
    return functools.partial(
        pl.kernel,
        out_type=jax.ShapeDtypeStruct((N_P, TW), jnp.float32),
        mesh=mesh,
        scratch_types=[
            pltpu.VMEM((N_P,), jnp.float32),
            pltpu.VMEM((16, 128), jnp.int32),
            pltpu.VMEM((16, 128), jnp.int32),
        ] + [pltpu.VMEM((128, TW), jnp.float32)] * 4
          + [pltpu.SemaphoreType.DMA] * 8
          + [pltpu.VMEM_SHARED((ACC_ROWS, TW), jnp.float32)],
        compiler_params=pltpu.CompilerParams(needs_layout_passes=False,
                                             use_tc_tiling_on_sc=False),
    )(functools.partial(_seg_body, TW))


def _seg_call(TW, table, keep, a, b):
    return _make_seg(TW)(table, keep, a, b)


def _lrelu(v):
    return jnp.where(v >= 0, v, 0.1 * v)


def _dot(a, b):
    return lax.dot_general(a, b, (((1,), (0,)), ((), ())),
                           preferred_element_type=jnp.float32)


def _conv_body(s_ref, cnt_ref, t_ref, k_ref, wl_ref, wr_ref, b_ref, wp_ref,
               xo_ref, gt_ref, h_ref, dinv_ref):
    srows = s_ref[...]
    c = cnt_ref[...][:, 0:1]
    xin = t_ref[...]
    mean = srows / jnp.maximum(c, 1.0)
    z = _dot(mean, wl_ref[...]) + _dot(xin, wr_ref[...]) + b_ref[...]
    xo = _lrelu(z)
    xo_ref[...] = xo
    hc = _dot(xo, wp_ref[...])[:, 0:1]
    kf = k_ref[...]
    deg = c + kf
    dinv = jnp.where(deg > 0, lax.rsqrt(jnp.maximum(deg, 1e-30)), 0.0)
    lane16 = lax.broadcasted_iota(jnp.int32, (BR, 16), 1)
    gt_ref[...] = jnp.where(lane16 == 0, dinv * hc, 0.0)
    h_ref[...] = hc
    dinv_ref[...] = dinv


def _score_body(e_ref, dinv_ref, h_ref, k_ref, bp_ref, out_ref):
    esum = e_ref[...][:, 0:1]
    dinv = dinv_ref[...]
    out_ref[...] = (dinv * esum
                    + dinv * dinv * k_ref[...] * h_ref[...] + bp_ref[0, 0])


def _pool_body(xo_ref, sc_ref, kc_ref, bc_ref, scR_ref, kR_ref, bR_ref,
               xn_ref, kn_ref, rs_ref, rm_ref, rc_ref):
    i = pl.program_id(0)
    sc = sc_ref[...]
    kc = kc_ref[...]
    bc = bc_ref[...]
    kR = kR_ref[...]
    bR = bR_ref[...]
    kp_col = jnp.zeros((BR, 1), jnp.float32)
    for g in range(NG):
        ckg = jnp.sum(jnp.where(bR == float(g), kR, 0.0))
        kpg = jnp.floor((ckg + 1.0) * 0.5)
        kp_col = kp_col + jnp.where(bc == float(g), kpg, 0.0)
    ii = i * BR + lax.broadcasted_iota(jnp.int32, (BR, 128), 0)

    def jstep(jc, acc):
        sj = scR_ref[jc].reshape(1, 128)
        kj = kR_ref[jc].reshape(1, 128)
        bj = bR_ref[jc].reshape(1, 128)
        jj = jc * 128 + lax.broadcasted_iota(jnp.int32, (BR, 128), 1)
        beat = (sj > sc) | ((sj == sc) & (jj < ii))
        m = (bj == bc) & (kj > 0.0)
        return acc + jnp.sum(jnp.where(beat & m, 1.0, 0.0), axis=1,
                             keepdims=True)

    rank = lax.fori_loop(0, JR, jstep, jnp.zeros((BR, 1), jnp.float32))
    kn = jnp.where((kc > 0.0) & (rank < kp_col), 1.0, 0.0)
    x_new = jnp.where(kn > 0.0, xo_ref[...] * jnp.tanh(sc), 0.0)
    xn_ref[...] = x_new
    kn_ref[...] = kn

    @pl.when(i == 0)
    def _init():
        rs_ref[...] = jnp.zeros((NG, 128), jnp.float32)
        rm_ref[...] = jnp.full((NG, 128), -3.4e38, jnp.float32)
        rc_ref[...] = jnp.zeros((NG, 128), jnp.float32)

    for g in range(NG):
        selg = bc == float(g)
        rs_ref[g:g + 1, :] += jnp.sum(jnp.where(selg, x_new, 0.0), axis=0,
                                      keepdims=True)
        rm_ref[g:g + 1, :] = jnp.maximum(
            rm_ref[g:g + 1, :],
            jnp.max(jnp.where(selg & (kn > 0.0), x_new, -3.4e38), axis=0,
                    keepdims=True))
        rc_ref[g:g + 1, :] += jnp.sum(jnp.where(selg, kn, 0.0), axis=0,
                                      keepdims=True)


def _ytab_body(d_ref, t_ref, y_ref, dinv_ref):
    degS = d_ref[...][:, 0:1]
    dinvS = jnp.where(degS > 0, lax.rsqrt(jnp.maximum(degS, 1e-30)), 0.0)
    y_ref[...] = dinvS * t_ref[...]
    dinv_ref[...] = dinvS


def _head_body(t_ref, S_ref, dinvS_ref, k_ref,
               rs1, rm1, rc1, rs2, rm2, rc2, rs3, rm3, rc3,
               wl1, bl1, wl2, bl2, wl3, bl3,
               mean_ref, ge_ref, lg_ref, acc_ref):
    i = pl.program_id(0)

    @pl.when(i == 0)
    def _init():
        acc_ref[0] = 0.0
        acc_ref[1] = 0.0

    S = S_ref[...]
    agg = dinvS_ref[...] * S
    info = t_ref[...] - agg
    sn = jnp.sum(jnp.abs(info), axis=1, keepdims=True)
    kf = k_ref[...]
    acc_ref[0] += jnp.sum(sn * kf)
    acc_ref[1] += jnp.sum(kf)

    @pl.when(i == NBLK - 1)
    def _final():
        mean_ref[...] = (acc_ref[0] / acc_ref[1])[None, None]

        def readout(rs, rm, rc):
            mn = rs[...] / jnp.maximum(rc[...], 1.0)
            mx = jnp.where(rm[...] > -1e37, rm[...], 0.0)
            return jnp.concatenate([mx, mn], axis=1)

        h = (_lrelu(readout(rs1, rm1, rc1))
             + _lrelu(readout(rs2, rm2, rc2))
             + _lrelu(readout(rs3, rm3, rc3)))
        ge_ref[...] = h
        h1 = _lrelu(_dot(h, wl1[...]) + bl1[...])
        h2 = _lrelu(_dot(h1, wl2[...]) + bl2[...])
        lg_ref[...] = _dot(h2, wl3[...]) + bl3[...]


_f32 = jnp.float32


def _tc_conv(part, cnt, t, kcol, Wl, Wr, brow, wp_pad):
    return pl.pallas_call(
        _conv_body,
        grid=(NBLK,),
        in_specs=[
            pl.BlockSpec((BR, 128), lambda i: (i, 0)),
            pl.BlockSpec((BR, 16), lambda i: (i, 0)),
            pl.BlockSpec((BR, 128), lambda i: (i, 0)),
            pl.BlockSpec((BR, 1), lambda i: (i, 0)),
            pl.BlockSpec((F, F), lambda i: (0, 0)),
            pl.BlockSpec((F, F), lambda i: (0, 0)),
            pl.BlockSpec((1, F), lambda i: (0, 0)),
            pl.BlockSpec((F, F), lambda i: (0, 0)),
        ],
        out_specs=[
            pl.BlockSpec((BR, F), lambda i: (i, 0)),
            pl.BlockSpec((BR, 16), lambda i: (i, 0)),
            pl.BlockSpec((BR, 1), lambda i: (i, 0)),
            pl.BlockSpec((BR, 1), lambda i: (i, 0)),
        ],
        out_shape=[
            jax.ShapeDtypeStruct((N_P, F), _f32),
            jax.ShapeDtypeStruct((N_P, 16), _f32),
            jax.ShapeDtypeStruct((N_P, 1), _f32),
            jax.ShapeDtypeStruct((N_P, 1), _f32),
        ],
    )(part, cnt, t, kcol, Wl, Wr, brow, wp_pad)


def _tc_score(epart, dinv, hcol, kcol, bp):
    return pl.pallas_call(
        _score_body,
        grid=(NBLK,),
        in_specs=[
            pl.BlockSpec((BR, 16), lambda i: (i, 0)),
            pl.BlockSpec((BR, 1), lambda i: (i, 0)),
            pl.BlockSpec((BR, 1), lambda i: (i, 0)),
            pl.BlockSpec((BR, 1), lambda i: (i, 0)),
            pl.BlockSpec((1, 1), lambda i: (0, 0)),
        ],
        out_specs=pl.BlockSpec((BR, 1), lambda i: (i, 0)),
        out_shape=jax.ShapeDtypeStruct((N_P, 1), _f32),
    )(epart, dinv, hcol, kcol, bp)


def _tc_pool(xo, scc, kcol, bcol, scR, kR, bR):
    return pl.pallas_call(
        _pool_body,
        grid=(NBLK,),
        in_specs=[
            pl.BlockSpec((BR, F), lambda i: (i, 0)),
            pl.BlockSpec((BR, 1), lambda i: (i, 0)),
            pl.BlockSpec((BR, 1), lambda i: (i, 0)),
            pl.BlockSpec((BR, 1), lambda i: (i, 0)),
            pl.BlockSpec((JR, 128), lambda i: (0, 0)),
            pl.BlockSpec((JR, 128), lambda i: (0, 0)),
            pl.BlockSpec((JR, 128), lambda i: (0, 0)),
        ],
        out_specs=[
            pl.BlockSpec((BR, 128), lambda i: (i, 0)),
            pl.BlockSpec((BR, 1), lambda i: (i, 0)),
            pl.BlockSpec((NG, 128), lambda i: (0, 0)),
            pl.BlockSpec((NG, 128), lambda i: (0, 0)),
            pl.BlockSpec((NG, 128), lambda i: (0, 0)),
        ],
        out_shape=[
            jax.ShapeDtypeStruct((N_P, 128), _f32),
            jax.ShapeDtypeStruct((N_P, 1), _f32),
            jax.ShapeDtypeStruct((NG, 128), _f32),
            jax.ShapeDtypeStruct((NG, 128), _f32),
            jax.ShapeDtypeStruct((NG, 128), _f32),
        ],
    )(xo, scc, kcol, bcol, scR, kR, bR)


def _tc_ytab(degpart, t):
    return pl.pallas_call(
        _ytab_body,
        grid=(NBLK,),
        in_specs=[
            pl.BlockSpec((BR, 16), lambda i: (i, 0)),
            pl.BlockSpec((BR, 128), lambda i: (i, 0)),
        ],
        out_specs=[
            pl.BlockSpec((BR, 128), lambda i: (i, 0)),
            pl.BlockSpec((BR, 1), lambda i: (i, 0)),
        ],
        out_shape=[
            jax.ShapeDtypeStruct((N_P, 128), _f32),
            jax.ShapeDtypeStruct((N_P, 1), _f32),
        ],
    )(degpart, t)


def _tc_head(t, Spart, dinvS, kcol, reads, WL1, bL1, WL2, bL2, WL3p, bL3p):
    full = lambda shp: pl.BlockSpec(shp, lambda i: (0, 0))
    rspecs = []
    rargs = []
    for rs, rm, rc in reads:
        rspecs += [full((NG, 128))] * 3
        rargs += [rs, rm, rc]
    return pl.pallas_call(
        _head_body,
        grid=(NBLK,),
        in_specs=[
            pl.BlockSpec((BR, 128), lambda i: (i, 0)),
            pl.BlockSpec((BR, 128), lambda i: (i, 0)),
            pl.BlockSpec((BR, 1), lambda i: (i, 0)),
            pl.BlockSpec((BR, 1), lambda i: (i, 0)),
        ] + rspecs + [
            full((2 * F, F)), full((1, F)),
            full((F, 64)), full((1, 64)),
            full((64, 128)), full((1, 128)),
        ],
        out_specs=[
            full((1, 1)), full((NG, 2 * F)), full((NG, 128)),
        ],
        out_shape=[
            jax.ShapeDtypeStruct((1, 1), _f32),
            jax.ShapeDtypeStruct((NG, 2 * F), _f32),
            jax.ShapeDtypeStruct((NG, 128), _f32),
        ],
        scratch_shapes=[pltpu.SMEM((2,), _f32)],
    )(t, Spart, dinvS, kcol, *rargs, WL1, bL1, WL2, bL2, WL3p, bL3p)


def kernel(x, edge_index, batch, W1l, b1, W1r, W2l, b2, W2r, W3l, b3, W3r,
           Wp1, bp1, Wp2, bp2, Wp3, bp3, WL1, bL1, WL2, bL2, WL3, bL3):
    f32 = jnp.float32
    src = edge_index[0]
    dst = edge_index[1]
    srcp = jnp.pad(src, (0, EP - E)).reshape(EPR, 128)
    dstp = jnp.pad(dst, (0, EP - E), constant_values=DUMMY).reshape(EPR, 128)
    t = jnp.pad(x, ((0, PAD), (0, 0)))
    lane16 = jnp.arange(16)
    ones_t16 = (jnp.where(lane16[None, :] == 0, 1.0, 0.0)
                * jnp.ones((N_P, 1), f32))
    batchf = jnp.pad(batch.astype(f32), (0, PAD), constant_values=15.0)
    keep = jnp.pad(jnp.ones((N,), f32), (0, PAD))
    bcol = batchf.reshape(N_P, 1)
    bR = batchf.reshape(JR, 128)

    weights = [(W1l, b1, W1r, Wp1, bp1), (W2l, b2, W2r, Wp2, bp2),
               (W3l, b3, W3r, Wp3, bp3)]
    reads = []
    for (Wl, b, Wr, Wp, bp) in weights:
        part = _seg_call(TW_BIG, t, keep, srcp, dstp)
        cnt = _seg_call(TW_SMALL, ones_t16, keep, srcp, dstp)
        kcol = keep.reshape(N_P, 1)
        xo, gt, hcol, dinv = _tc_conv(
            part, cnt, t, kcol, Wl, Wr, b.reshape(1, F),
            jnp.pad(Wp, ((0, 0), (0, F - 1))))
        epart = _seg_call(TW_SMALL, gt, keep, srcp, dstp)
        score = _tc_score(epart, dinv, hcol, kcol, bp.reshape(1, 1))
        t, kn, rs, rm, rc = _tc_pool(
            xo, score, kcol, bcol, score.reshape(JR, 128),
            keep.reshape(JR, 128), bR)
        keep = kn.reshape(N_P)
        reads.append((rs, rm, rc))

    degpart = _seg_call(TW_SMALL, ones_t16, keep, dstp, srcp)
    yt, dinvS = _tc_ytab(degpart, t)
    Spart = _seg_call(TW_BIG, yt, keep, srcp, dstp)
    mean1, ge, lgp = _tc_head(
        t, Spart, dinvS, keep.reshape(N_P, 1), reads,
        WL1, bL1.reshape(1, F), WL2, bL2.reshape(1, 64),
        jnp.pad(WL3, ((0, 0), (0, 128 - 30))),
        jnp.pad(bL3, (0, 128 - 30)).reshape(1, 128))
    return lgp[:NG, :30], mean1[0, 0], ge


# speedup vs baseline: 2.4857x; 2.4857x over previous
import functools

import jax
import jax.numpy as jnp
from jax import lax
from jax.experimental import pallas as pl
from jax.experimental.pallas import tpu as pltpu
from jax.experimental.pallas import tpu_sc as plsc

N = 10000
E = 320000
F = 128
NG = 16
N_P = 10112
PAD = N_P - N
DUMMY = N
NW = 32
CH = 80
EW = CH * 128
EP = EW * NW
EPR = EP // 128
BR = 632
NBLK = N_P // BR
JR = N_P // 128
TW_BIG = 128
TW_SMALL = 16
NHALF = N_P // 2
ACC_ROWS = 5120
DUMMY_L = NHALF
CAP = EP // 16 + 32


def _seg_body(TW, table_h, keep_h, a_h, b_h, out_h, keep_v, ab, bb,
              ca_f, cb_f, r0, r1, g0, g1, s0, s1, acc):
    cid = lax.axis_index("c")
    sid = lax.axis_index("s")
    lo = cid * NHALF

    pltpu.sync_copy(keep_h, keep_v)

    @pl.loop(0, 128)
    def _z(r):
        for kk in range(TW // 16):
            r0[r, pl.ds(kk * 16, 16)] = jnp.zeros((16,), jnp.float32)

    zbase = sid * (ACC_ROWS // 16)
    for q in range(2):
        pltpu.sync_copy(r0, acc.at[pl.ds(zbase + q * 128, 128)])
    pltpu.sync_copy(r0.at[pl.ds(0, 64)], acc.at[pl.ds(zbase + 256, 64)])
    plsc.subcore_barrier()

    row0 = sid * (EPR // 16)

    def _m(m, cnt):
        mrow = row0 + m * 16
        pltpu.sync_copy(a_h.at[pl.ds(mrow, 16)], ab)
        pltpu.sync_copy(b_h.at[pl.ds(mrow, 16)], bb)

        def _j(j, cnt):
            def _k(k, cnt):
                a16 = ab[j, pl.ds(k * 16, 16)]
                b16 = bb[j, pl.ds(k * 16, 16)]
                em = (plsc.load_gather(keep_v, [a16])
                      * plsc.load_gather(keep_v, [b16]))
                dl = b16 - lo
                live = (em > 0.0) & (dl >= 0) & (dl < NHALF)
                plsc.store_compressed(ca_f.at[pl.ds(cnt, 16)], a16,
                                      mask=live)
                plsc.store_compressed(cb_f.at[pl.ds(cnt, 16)], dl,
                                      mask=live)
                return cnt + jnp.max(
                    plsc.all_reduce_population_count(live))

            return pl.loop(0, 8, init_carry=cnt)(_k)

        return pl.loop(0, 16, init_carry=cnt)(_j)

    cnt = pl.loop(0, EPR // 256, init_carry=jnp.int32(0))(_m)

    pad_to = ((cnt + 127) // 128) * 128

    @pl.loop(0, 8)
    def _t(p):
        off = cnt + p * 16

        @pl.when(off < pad_to)
        def _():
            ca_f[pl.ds(off, 16)] = jnp.zeros((16,), jnp.int32)
            cb_f[pl.ds(off, 16)] = jnp.full((16,), DUMMY_L, jnp.int32)

    npair = pad_to // 256
    rem = pad_to - npair * 256

    @pl.loop(0, npair)
    def _p(p):
        c0 = p * 256
        c1 = c0 + 128
        pltpu.async_copy(table_h.at[ca_f.at[pl.ds(c0, 128)]], r0, g0)
        pltpu.async_copy(table_h.at[ca_f.at[pl.ds(c1, 128)]], r1, g1)
        pltpu.make_async_copy(table_h.at[ca_f.at[pl.ds(c0, 128)]], r0,
                              g0).wait()
        pltpu.async_copy(r0, acc.at[cb_f.at[pl.ds(c0, 128)]], s0, add=True)
        pltpu.make_async_copy(table_h.at[ca_f.at[pl.ds(c1, 128)]], r1,
                              g1).wait()
        pltpu.async_copy(r1, acc.at[cb_f.at[pl.ds(c1, 128)]], s1, add=True)
        pltpu.make_async_copy(r0, acc.at[cb_f.at[pl.ds(c0, 128)]],
                              s0).wait()
        pltpu.make_async_copy(r1, acc.at[cb_f.at[pl.ds(c1, 128)]],
                              s1).wait()

    @pl.when(rem > 0)
    def _tail():
        c0 = npair * 256
        pltpu.sync_copy(table_h.at[ca_f.at[pl.ds(c0, 128)]], r0)
        pltpu.sync_copy(r0, acc.at[cb_f.at[pl.ds(c0, 128)]], add=True)

    plsc.subcore_barrier()

    @pl.when(sid < 8)
    def _out():
        obase = sid * BR
        for q in range(4):
            pltpu.sync_copy(acc.at[pl.ds(obase + q * 128, 128)],
                            out_h.at[pl.ds(lo + obase + q * 128, 128)])
        pltpu.sync_copy(acc.at[pl.ds(obase + 512, 120)],
                        out_h.at[pl.ds(lo + obase + 512, 120)])


def _make_seg(TW):
    mesh = plsc.VectorSubcoreMesh(core_axis_name="c", subcore_axis_name="s")
    return functools.partial(
        pl.kernel,
        out_type=jax.ShapeDtypeStruct((N_P, TW), jnp.float32),
        mesh=mesh,
        scratch_types=[
            pltpu.VMEM((N_P,), jnp.float32),
            pltpu.VMEM((16, 128), jnp.int32),
            pltpu.VMEM((16, 128), jnp.int32),
            pltpu.VMEM((CAP,), jnp.int32),
            pltpu.VMEM((CAP,), jnp.int32),
        ] + [pltpu.VMEM((128, TW), jnp.float32)] * 2
          + [pltpu.SemaphoreType.DMA] * 4
          + [pltpu.VMEM_SHARED((ACC_ROWS, TW), jnp.float32)],
        compiler_params=pltpu.CompilerParams(needs_layout_passes=False,
                                             use_tc_tiling_on_sc=False),
    )(functools.partial(_seg_body, TW))


def _seg_call(TW, table, keep, a, b):
    return _make_seg(TW)(table, keep, a, b)


def _lrelu(v):
    return jnp.where(v >= 0, v, 0.1 * v)


def _dot(a, b):
    return lax.dot_general(a, b, (((1,), (0,)), ((), ())),
                           preferred_element_type=jnp.float32)


def _conv_body(s_ref, cnt_ref, t_ref, k_ref, wl_ref, wr_ref, b_ref, wp_ref,
               xo_ref, gt_ref, h_ref, dinv_ref):
    srows = s_ref[...]
    c = cnt_ref[...][:, 0:1]
    xin = t_ref[...]
    mean = srows / jnp.maximum(c, 1.0)
    z = _dot(mean, wl_ref[...]) + _dot(xin, wr_ref[...]) + b_ref[...]
    xo = _lrelu(z)
    xo_ref[...] = xo
    hc = _dot(xo, wp_ref[...])[:, 0:1]
    kf = k_ref[...]
    deg = c + kf
    dinv = jnp.where(deg > 0, lax.rsqrt(jnp.maximum(deg, 1e-30)), 0.0)
    lane16 = lax.broadcasted_iota(jnp.int32, (BR, 16), 1)
    gt_ref[...] = jnp.where(lane16 == 0, dinv * hc, 0.0)
    h_ref[...] = hc
    dinv_ref[...] = dinv


def _score_body(e_ref, dinv_ref, h_ref, k_ref, bp_ref, out_ref):
    esum = e_ref[...][:, 0:1]
    dinv = dinv_ref[...]
    out_ref[...] = (dinv * esum
                    + dinv * dinv * k_ref[...] * h_ref[...] + bp_ref[0, 0])


def _pool_body(xo_ref, sc_ref, kc_ref, bc_ref, scR_ref, kR_ref, bR_ref,
               xn_ref, kn_ref, rs_ref, rm_ref, rc_ref):
    i = pl.program_id(0)
    sc = sc_ref[...]
    kc = kc_ref[...]
    bc = bc_ref[...]
    kR = kR_ref[...]
    bR = bR_ref[...]
    kp_col = jnp.zeros((BR, 1), jnp.float32)
    for g in range(NG):
        ckg = jnp.sum(jnp.where(bR == float(g), kR, 0.0))
        kpg = jnp.floor((ckg + 1.0) * 0.5)
        kp_col = kp_col + jnp.where(bc == float(g), kpg, 0.0)
    ii = i * BR + lax.broadcasted_iota(jnp.int32, (BR, 128), 0)

    def jstep(jc, acc):
        sj = scR_ref[jc].reshape(1, 128)
        kj = kR_ref[jc].reshape(1, 128)
        bj = bR_ref[jc].reshape(1, 128)
        jj = jc * 128 + lax.broadcasted_iota(jnp.int32, (BR, 128), 1)
        beat = (sj > sc) | ((sj == sc) & (jj < ii))
        m = (bj == bc) & (kj > 0.0)
        return acc + jnp.sum(jnp.where(beat & m, 1.0, 0.0), axis=1,
                             keepdims=True)

    rank = lax.fori_loop(0, JR, jstep, jnp.zeros((BR, 1), jnp.float32))
    kn = jnp.where((kc > 0.0) & (rank < kp_col), 1.0, 0.0)
    x_new = jnp.where(kn > 0.0, xo_ref[...] * jnp.tanh(sc), 0.0)
    xn_ref[...] = x_new
    kn_ref[...] = kn

    @pl.when(i == 0)
    def _init():
        rs_ref[...] = jnp.zeros((NG, 128), jnp.float32)
        rm_ref[...] = jnp.full((NG, 128), -3.4e38, jnp.float32)
        rc_ref[...] = jnp.zeros((NG, 128), jnp.float32)

    for g in range(NG):
        selg = bc == float(g)
        rs_ref[g:g + 1, :] += jnp.sum(jnp.where(selg, x_new, 0.0), axis=0,
                                      keepdims=True)
        rm_ref[g:g + 1, :] = jnp.maximum(
            rm_ref[g:g + 1, :],
            jnp.max(jnp.where(selg & (kn > 0.0), x_new, -3.4e38), axis=0,
                    keepdims=True))
        rc_ref[g:g + 1, :] += jnp.sum(jnp.where(selg, kn, 0.0), axis=0,
                                      keepdims=True)


def _ytab_body(d_ref, t_ref, y_ref, dinv_ref):
    degS = d_ref[...][:, 0:1]
    dinvS = jnp.where(degS > 0, lax.rsqrt(jnp.maximum(degS, 1e-30)), 0.0)
    y_ref[...] = dinvS * t_ref[...]
    dinv_ref[...] = dinvS


def _head_body(t_ref, S_ref, dinvS_ref, k_ref,
               rs1, rm1, rc1, rs2, rm2, rc2, rs3, rm3, rc3,
               wl1, bl1, wl2, bl2, wl3, bl3,
               mean_ref, ge_ref, lg_ref, acc_ref):
    i = pl.program_id(0)

    @pl.when(i == 0)
    def _init():
        acc_ref[0] = 0.0
        acc_ref[1] = 0.0

    S = S_ref[...]
    agg = dinvS_ref[...] * S
    info = t_ref[...] - agg
    sn = jnp.sum(jnp.abs(info), axis=1, keepdims=True)
    kf = k_ref[...]
    acc_ref[0] += jnp.sum(sn * kf)
    acc_ref[1] += jnp.sum(kf)

    @pl.when(i == NBLK - 1)
    def _final():
        mean_ref[...] = (acc_ref[0] / acc_ref[1])[None, None]

        def readout(rs, rm, rc):
            mn = rs[...] / jnp.maximum(rc[...], 1.0)
            mx = jnp.where(rm[...] > -1e37, rm[...], 0.0)
            return jnp.concatenate([mx, mn], axis=1)

        h = (_lrelu(readout(rs1, rm1, rc1))
             + _lrelu(readout(rs2, rm2, rc2))
             + _lrelu(readout(rs3, rm3, rc3)))
        ge_ref[...] = h
        h1 = _lrelu(_dot(h, wl1[...]) + bl1[...])
        h2 = _lrelu(_dot(h1, wl2[...]) + bl2[...])
        lg_ref[...] = _dot(h2, wl3[...]) + bl3[...]


_f32 = jnp.float32


def _tc_conv(part, cnt, t, kcol, Wl, Wr, brow, wp_pad):
    return pl.pallas_call(
        _conv_body,
        grid=(NBLK,),
        in_specs=[
            pl.BlockSpec((BR, 128), lambda i: (i, 0)),
            pl.BlockSpec((BR, 16), lambda i: (i, 0)),
            pl.BlockSpec((BR, 128), lambda i: (i, 0)),
            pl.BlockSpec((BR, 1), lambda i: (i, 0)),
            pl.BlockSpec((F, F), lambda i: (0, 0)),
            pl.BlockSpec((F, F), lambda i: (0, 0)),
            pl.BlockSpec((1, F), lambda i: (0, 0)),
            pl.BlockSpec((F, F), lambda i: (0, 0)),
        ],
        out_specs=[
            pl.BlockSpec((BR, F), lambda i: (i, 0)),
            pl.BlockSpec((BR, 16), lambda i: (i, 0)),
            pl.BlockSpec((BR, 1), lambda i: (i, 0)),
            pl.BlockSpec((BR, 1), lambda i: (i, 0)),
        ],
        out_shape=[
            jax.ShapeDtypeStruct((N_P, F), _f32),
            jax.ShapeDtypeStruct((N_P, 16), _f32),
            jax.ShapeDtypeStruct((N_P, 1), _f32),
            jax.ShapeDtypeStruct((N_P, 1), _f32),
        ],
    )(part, cnt, t, kcol, Wl, Wr, brow, wp_pad)


def _tc_score(epart, dinv, hcol, kcol, bp):
    return pl.pallas_call(
        _score_body,
        grid=(NBLK,),
        in_specs=[
            pl.BlockSpec((BR, 16), lambda i: (i, 0)),
            pl.BlockSpec((BR, 1), lambda i: (i, 0)),
            pl.BlockSpec((BR, 1), lambda i: (i, 0)),
            pl.BlockSpec((BR, 1), lambda i: (i, 0)),
            pl.BlockSpec((1, 1), lambda i: (0, 0)),
        ],
        out_specs=pl.BlockSpec((BR, 1), lambda i: (i, 0)),
        out_shape=jax.ShapeDtypeStruct((N_P, 1), _f32),
    )(epart, dinv, hcol, kcol, bp)


def _tc_pool(xo, scc, kcol, bcol, scR, kR, bR):
    return pl.pallas_call(
        _pool_body,
        grid=(NBLK,),
        in_specs=[
            pl.BlockSpec((BR, F), lambda i: (i, 0)),
            pl.BlockSpec((BR, 1), lambda i: (i, 0)),
            pl.BlockSpec((BR, 1), lambda i: (i, 0)),
            pl.BlockSpec((BR, 1), lambda i: (i, 0)),
            pl.BlockSpec((JR, 128), lambda i: (0, 0)),
            pl.BlockSpec((JR, 128), lambda i: (0, 0)),
            pl.BlockSpec((JR, 128), lambda i: (0, 0)),
        ],
        out_specs=[
            pl.BlockSpec((BR, 128), lambda i: (i, 0)),
            pl.BlockSpec((BR, 1), lambda i: (i, 0)),
            pl.BlockSpec((NG, 128), lambda i: (0, 0)),
            pl.BlockSpec((NG, 128), lambda i: (0, 0)),
            pl.BlockSpec((NG, 128), lambda i: (0, 0)),
        ],
        out_shape=[
            jax.ShapeDtypeStruct((N_P, 128), _f32),
            jax.ShapeDtypeStruct((N_P, 1), _f32),
            jax.ShapeDtypeStruct((NG, 128), _f32),
            jax.ShapeDtypeStruct((NG, 128), _f32),
            jax.ShapeDtypeStruct((NG, 128), _f32),
        ],
    )(xo, scc, kcol, bcol, scR, kR, bR)


def _tc_ytab(degpart, t):
    return pl.pallas_call(
        _ytab_body,
        grid=(NBLK,),
        in_specs=[
            pl.BlockSpec((BR, 16), lambda i: (i, 0)),
            pl.BlockSpec((BR, 128), lambda i: (i, 0)),
        ],
        out_specs=[
            pl.BlockSpec((BR, 128), lambda i: (i, 0)),
            pl.BlockSpec((BR, 1), lambda i: (i, 0)),
        ],
        out_shape=[
            jax.ShapeDtypeStruct((N_P, 128), _f32),
            jax.ShapeDtypeStruct((N_P, 1), _f32),
        ],
    )(degpart, t)


def _tc_head(t, Spart, dinvS, kcol, reads, WL1, bL1, WL2, bL2, WL3p, bL3p):
    full = lambda shp: pl.BlockSpec(shp, lambda i: (0, 0))
    rspecs = []
    rargs = []
    for rs, rm, rc in reads:
        rspecs += [full((NG, 128))] * 3
        rargs += [rs, rm, rc]
    return pl.pallas_call(
        _head_body,
        grid=(NBLK,),
        in_specs=[
            pl.BlockSpec((BR, 128), lambda i: (i, 0)),
            pl.BlockSpec((BR, 128), lambda i: (i, 0)),
            pl.BlockSpec((BR, 1), lambda i: (i, 0)),
            pl.BlockSpec((BR, 1), lambda i: (i, 0)),
        ] + rspecs + [
            full((2 * F, F)), full((1, F)),
            full((F, 64)), full((1, 64)),
            full((64, 128)), full((1, 128)),
        ],
        out_specs=[
            full((1, 1)), full((NG, 2 * F)), full((NG, 128)),
        ],
        out_shape=[
            jax.ShapeDtypeStruct((1, 1), _f32),
            jax.ShapeDtypeStruct((NG, 2 * F), _f32),
            jax.ShapeDtypeStruct((NG, 128), _f32),
        ],
        scratch_shapes=[pltpu.SMEM((2,), _f32)],
    )(t, Spart, dinvS, kcol, *rargs, WL1, bL1, WL2, bL2, WL3p, bL3p)


def kernel(x, edge_index, batch, W1l, b1, W1r, W2l, b2, W2r, W3l, b3, W3r,
           Wp1, bp1, Wp2, bp2, Wp3, bp3, WL1, bL1, WL2, bL2, WL3, bL3):
    f32 = jnp.float32
    src = edge_index[0]
    dst = edge_index[1]
    srcp = jnp.pad(src, (0, EP - E)).reshape(EPR, 128)
    dstp = jnp.pad(dst, (0, EP - E), constant_values=DUMMY).reshape(EPR, 128)
    t = jnp.pad(x, ((0, PAD), (0, 0)))
    lane16 = jnp.arange(16)
    ones_t16 = (jnp.where(lane16[None, :] == 0, 1.0, 0.0)
                * jnp.ones((N_P, 1), f32))
    batchf = jnp.pad(batch.astype(f32), (0, PAD), constant_values=15.0)
    keep = jnp.pad(jnp.ones((N,), f32), (0, PAD))
    bcol = batchf.reshape(N_P, 1)
    bR = batchf.reshape(JR, 128)

    weights = [(W1l, b1, W1r, Wp1, bp1), (W2l, b2, W2r, Wp2, bp2),
               (W3l, b3, W3r, Wp3, bp3)]
    reads = []
    for (Wl, b, Wr, Wp, bp) in weights:
        part = _seg_call(TW_BIG, t, keep, srcp, dstp)
        cnt = _seg_call(TW_SMALL, ones_t16, keep, srcp, dstp)
        kcol = keep.reshape(N_P, 1)
        xo, gt, hcol, dinv = _tc_conv(
            part, cnt, t, kcol, Wl, Wr, b.reshape(1, F),
            jnp.pad(Wp, ((0, 0), (0, F - 1))))
        epart = _seg_call(TW_SMALL, gt, keep, srcp, dstp)
        score = _tc_score(epart, dinv, hcol, kcol, bp.reshape(1, 1))
        t, kn, rs, rm, rc = _tc_pool(
            xo, score, kcol, bcol, score.reshape(JR, 128),
            keep.reshape(JR, 128), bR)
        keep = kn.reshape(N_P)
        reads.append((rs, rm, rc))

    degpart = _seg_call(TW_SMALL, ones_t16, keep, dstp, srcp)
    yt, dinvS = _tc_ytab(degpart, t)
    Spart = _seg_call(TW_BIG, yt, keep, srcp, dstp)
    mean1, ge, lgp = _tc_head(
        t, Spart, dinvS, keep.reshape(N_P, 1), reads,
        WL1, bL1.reshape(1, F), WL2, bL2.reshape(1, 64),
        jnp.pad(WL3, ((0, 0), (0, 128 - 30))),
        jnp.pad(bL3, (0, 128 - 30)).reshape(1, 128))
    return lgp[:NG, :30], mean1[0, 0], ge


# speedup vs baseline: 2.6818x; 1.0789x over previous
import functools

import jax
import jax.numpy as jnp
from jax import lax
from jax.experimental import pallas as pl
from jax.experimental.pallas import tpu as pltpu
from jax.experimental.pallas import tpu_sc as plsc

N = 10000
E = 320000
F = 128
NG = 16
N_P = 10112
PAD = N_P - N
DUMMY = N
NW = 32
CH = 80
EW = CH * 128
EP = EW * NW
EPR = EP // 128
BR = 632
NBLK = N_P // BR
JR = N_P // 128
TW_BIG = 128
TW_SMALL = 16
NHALF = N_P // 2
ACC_ROWS = 5120
DUMMY_L = NHALF
CAP = EP // 16 + 32


def _seg_body(TW, table_h, keep_h, a_h, b_h, *refs):
    with_cnt = TW == 128
    if with_cnt:
        (out_h, out16_h, keep_v, ab, bb, cpk, ia, ib, r0, r1, ones16,
         g0, g1, s0, s1, t0, t1, acc, acc16) = refs
    else:
        (out_h, keep_v, ab, bb, cpk, ia, ib, r0, r1,
         g0, g1, s0, s1, acc) = refs
    cid = lax.axis_index("c")
    sid = lax.axis_index("s")
    lo = cid * NHALF

    pltpu.sync_copy(keep_h, keep_v)

    @pl.loop(0, 128)
    def _z(r):
        for kk in range(TW // 16):
            r0[r, pl.ds(kk * 16, 16)] = jnp.zeros((16,), jnp.float32)

    zbase = sid * (ACC_ROWS // 16)
    for q in range(2):
        pltpu.sync_copy(r0, acc.at[pl.ds(zbase + q * 128, 128)])
    pltpu.sync_copy(r0.at[pl.ds(0, 64)], acc.at[pl.ds(zbase + 256, 64)])
    if with_cnt:
        @pl.loop(0, 128)
        def _z16(r):
            ones16[r, pl.ds(0, 16)] = jnp.zeros((16,), jnp.float32)

        for q in range(2):
            pltpu.sync_copy(ones16, acc16.at[pl.ds(zbase + q * 128, 128)])
        pltpu.sync_copy(ones16.at[pl.ds(0, 64)],
                        acc16.at[pl.ds(zbase + 256, 64)])
        col0 = (lax.broadcasted_iota(jnp.int32, (16,), 0) == 0
                ).astype(jnp.float32)

        @pl.loop(0, 128)
        def _o16(r):
            ones16[r, pl.ds(0, 16)] = col0

    plsc.subcore_barrier()

    row0 = sid * (EPR // 16)

    def _m(m, cnt):
        mrow = row0 + m * 16
        pltpu.sync_copy(a_h.at[pl.ds(mrow, 16)], ab)
        pltpu.sync_copy(b_h.at[pl.ds(mrow, 16)], bb)

        def _j(j, cnt):
            def _k(k, cnt):
                a16 = ab[j, pl.ds(k * 16, 16)]
                b16 = bb[j, pl.ds(k * 16, 16)]
                em = (plsc.load_gather(keep_v, [a16])
                      * plsc.load_gather(keep_v, [b16]))
                dl = b16 - lo
                live = (em > 0.0) & (dl >= 0) & (dl < NHALF)
                plsc.store_compressed(
                    cpk.at[pl.ds(cnt, 16)],
                    a16 | lax.shift_left(dl, 14), mask=live)
                return cnt + jnp.max(
                    plsc.all_reduce_population_count(live))

            return pl.loop(0, 8, init_carry=cnt)(_k)

        return pl.loop(0, 16, init_carry=cnt)(_j)

    cnt = pl.loop(0, EPR // 256, init_carry=jnp.int32(0))(_m)

    pad_to = ((cnt + 127) // 128) * 128

    @pl.loop(0, 8)
    def _t(p):
        off = cnt + p * 16

        @pl.when(off < pad_to)
        def _():
            cpk[pl.ds(off, 16)] = jnp.full((16,), DUMMY_L << 14, jnp.int32)

    npair = pad_to // 256
    rem = pad_to - npair * 256

    def _unpack(c0, qb):
        @pl.loop(0, 8)
        def _u(k):
            v = cpk[pl.ds(c0 + k * 16, 16)]
            ia[qb, pl.ds(k * 16, 16)] = v & 0x3FFF
            ib[qb, pl.ds(k * 16, 16)] = lax.shift_right_logical(v, 14)

    def _start(qb, rb, gs):
        pltpu.async_copy(table_h.at[ia.at[qb]], rb, gs)

    def _chunk(qb, rb, gs, ss, ts):
        pltpu.make_async_copy(table_h.at[ia.at[qb]], rb, gs).wait()
        pltpu.async_copy(rb, acc.at[ib.at[qb]], ss, add=True)
        if with_cnt:
            pltpu.async_copy(ones16, acc16.at[ib.at[qb]], ts, add=True)

    def _drain(qb, rb, ss, ts):
        pltpu.make_async_copy(rb, acc.at[ib.at[qb]], ss).wait()
        if with_cnt:
            pltpu.make_async_copy(ones16, acc16.at[ib.at[qb]], ts).wait()

    @pl.loop(0, npair)
    def _p(p):
        c0 = p * 256
        _unpack(c0, 0)
        _unpack(c0 + 128, 1)
        _start(0, r0, g0)
        _start(1, r1, g1)
        _chunk(0, r0, g0, s0, with_cnt and t0)
        _chunk(1, r1, g1, s1, with_cnt and t1)
        _drain(0, r0, s0, with_cnt and t0)
        _drain(1, r1, s1, with_cnt and t1)

    @pl.when(rem > 0)
    def _tail():
        _unpack(npair * 256, 0)
        _start(0, r0, g0)
        _chunk(0, r0, g0, s0, with_cnt and t0)
        _drain(0, r0, s0, with_cnt and t0)

    plsc.subcore_barrier()

    @pl.when(sid < 8)
    def _out():
        obase = sid * BR
        for q in range(4):
            pltpu.sync_copy(acc.at[pl.ds(obase + q * 128, 128)],
                            out_h.at[pl.ds(lo + obase + q * 128, 128)])
        pltpu.sync_copy(acc.at[pl.ds(obase + 512, 120)],
                        out_h.at[pl.ds(lo + obase + 512, 120)])
        if with_cnt:
            for q in range(4):
                pltpu.sync_copy(
                    acc16.at[pl.ds(obase + q * 128, 128)],
                    out16_h.at[pl.ds(lo + obase + q * 128, 128)])
            pltpu.sync_copy(acc16.at[pl.ds(obase + 512, 120)],
                            out16_h.at[pl.ds(lo + obase + 512, 120)])


def _make_seg(TW):
    mesh = plsc.VectorSubcoreMesh(core_axis_name="c", subcore_axis_name="s")
    with_cnt = TW == 128
    out_type = jax.ShapeDtypeStruct((N_P, TW), jnp.float32)
    if with_cnt:
        out_type = [out_type,
                    jax.ShapeDtypeStruct((N_P, 16), jnp.float32)]
    scratch = [
        pltpu.VMEM((N_P,), jnp.float32),
        pltpu.VMEM((16, 128), jnp.int32),
        pltpu.VMEM((16, 128), jnp.int32),
        pltpu.VMEM((CAP,), jnp.int32),
        pltpu.VMEM((2, 128), jnp.int32),
        pltpu.VMEM((2, 128), jnp.int32),
        pltpu.VMEM((128, TW), jnp.float32),
        pltpu.VMEM((128, TW), jnp.float32),
    ]
    if with_cnt:
        scratch += [pltpu.VMEM((128, 16), jnp.float32)]
        scratch += [pltpu.SemaphoreType.DMA] * 6
        scratch += [pltpu.VMEM_SHARED((ACC_ROWS, TW), jnp.float32),
                    pltpu.VMEM_SHARED((ACC_ROWS, 16), jnp.float32)]
    else:
        scratch += [pltpu.SemaphoreType.DMA] * 4
        scratch += [pltpu.VMEM_SHARED((ACC_ROWS, TW), jnp.float32)]
    return functools.partial(
        pl.kernel,
        out_type=out_type,
        mesh=mesh,
        scratch_types=scratch,
        compiler_params=pltpu.CompilerParams(needs_layout_passes=False,
                                             use_tc_tiling_on_sc=False),
    )(functools.partial(_seg_body, TW))


def _seg_call(TW, table, keep, a, b):
    return _make_seg(TW)(table, keep, a, b)


def _lrelu(v):
    return jnp.where(v >= 0, v, 0.1 * v)


def _dot(a, b):
    return lax.dot_general(a, b, (((1,), (0,)), ((), ())),
                           preferred_element_type=jnp.float32)


def _conv_body(s_ref, cnt_ref, t_ref, k_ref, wl_ref, wr_ref, b_ref, wp_ref,
               xo_ref, gt_ref, h_ref, dinv_ref):
    srows = s_ref[...]
    c = cnt_ref[...][:, 0:1]
    xin = t_ref[...]
    mean = srows / jnp.maximum(c, 1.0)
    z = _dot(mean, wl_ref[...]) + _dot(xin, wr_ref[...]) + b_ref[...]
    xo = _lrelu(z)
    xo_ref[...] = xo
    hc = _dot(xo, wp_ref[...])[:, 0:1]
    kf = k_ref[...]
    deg = c + kf
    dinv = jnp.where(deg > 0, lax.rsqrt(jnp.maximum(deg, 1e-30)), 0.0)
    lane16 = lax.broadcasted_iota(jnp.int32, (BR, 16), 1)
    gt_ref[...] = jnp.where(lane16 == 0, dinv * hc, 0.0)
    h_ref[...] = hc
    dinv_ref[...] = dinv


def _pool_body(xo_ref, e_ref, dinv_ref, h_ref, kc_ref, bc_ref,
               er_ref, dr_ref, hr_ref, kR_ref, bR_ref, bp_ref,
               xn_ref, kn_ref, rs_ref, rm_ref, rc_ref):
    i = pl.program_id(0)
    bp = bp_ref[0, 0]
    dinv = dinv_ref[...]
    sc = (dinv * e_ref[...][:, 0:1]
          + dinv * dinv * kc_ref[...] * h_ref[...] + bp)
    kc = kc_ref[...]
    bc = bc_ref[...]
    kR = kR_ref[...]
    bR = bR_ref[...]
    kp_col = jnp.zeros((BR, 1), jnp.float32)
    for g in range(NG):
        ckg = jnp.sum(jnp.where(bR == float(g), kR, 0.0))
        kpg = jnp.floor((ckg + 1.0) * 0.5)
        kp_col = kp_col + jnp.where(bc == float(g), kpg, 0.0)
    ii = i * BR + lax.broadcasted_iota(jnp.int32, (BR, 128), 0)

    def jstep(jc, acc):
        dj = dr_ref[jc].reshape(1, 128)
        kj = kR_ref[jc].reshape(1, 128)
        sj = (dj * er_ref[jc].reshape(1, 128)
              + dj * dj * kj * hr_ref[jc].reshape(1, 128) + bp)
        bj = bR_ref[jc].reshape(1, 128)
        jj = jc * 128 + lax.broadcasted_iota(jnp.int32, (BR, 128), 1)
        beat = (sj > sc) | ((sj == sc) & (jj < ii))
        m = (bj == bc) & (kj > 0.0)
        return acc + jnp.sum(jnp.where(beat & m, 1.0, 0.0), axis=1,
                             keepdims=True)

    rank = lax.fori_loop(0, JR, jstep, jnp.zeros((BR, 1), jnp.float32))
    kn = jnp.where((kc > 0.0) & (rank < kp_col), 1.0, 0.0)
    x_new = jnp.where(kn > 0.0, xo_ref[...] * jnp.tanh(sc), 0.0)
    xn_ref[...] = x_new
    kn_ref[...] = kn

    @pl.when(i == 0)
    def _init():
        rs_ref[...] = jnp.zeros((NG, 128), jnp.float32)
        rm_ref[...] = jnp.full((NG, 128), -3.4e38, jnp.float32)
        rc_ref[...] = jnp.zeros((NG, 128), jnp.float32)

    for g in range(NG):
        selg = bc == float(g)
        rs_ref[g:g + 1, :] += jnp.sum(jnp.where(selg, x_new, 0.0), axis=0,
                                      keepdims=True)
        rm_ref[g:g + 1, :] = jnp.maximum(
            rm_ref[g:g + 1, :],
            jnp.max(jnp.where(selg & (kn > 0.0), x_new, -3.4e38), axis=0,
                    keepdims=True))
        rc_ref[g:g + 1, :] += jnp.sum(jnp.where(selg, kn, 0.0), axis=0,
                                      keepdims=True)


def _ytab_body(d_ref, t_ref, y_ref, dinv_ref):
    degS = d_ref[...][:, 0:1]
    dinvS = jnp.where(degS > 0, lax.rsqrt(jnp.maximum(degS, 1e-30)), 0.0)
    y_ref[...] = dinvS * t_ref[...]
    dinv_ref[...] = dinvS


def _head_body(t_ref, S_ref, dinvS_ref, k_ref,
               rs1, rm1, rc1, rs2, rm2, rc2, rs3, rm3, rc3,
               wl1, bl1, wl2, bl2, wl3, bl3,
               mean_ref, ge_ref, lg_ref, acc_ref):
    i = pl.program_id(0)

    @pl.when(i == 0)
    def _init():
        acc_ref[0] = 0.0
        acc_ref[1] = 0.0

    S = S_ref[...]
    agg = dinvS_ref[...] * S
    info = t_ref[...] - agg
    sn = jnp.sum(jnp.abs(info), axis=1, keepdims=True)
    kf = k_ref[...]
    acc_ref[0] += jnp.sum(sn * kf)
    acc_ref[1] += jnp.sum(kf)

    @pl.when(i == NBLK - 1)
    def _final():
        mean_ref[...] = (acc_ref[0] / acc_ref[1])[None, None]

        def readout(rs, rm, rc):
            mn = rs[...] / jnp.maximum(rc[...], 1.0)
            mx = jnp.where(rm[...] > -1e37, rm[...], 0.0)
            return jnp.concatenate([mx, mn], axis=1)

        h = (_lrelu(readout(rs1, rm1, rc1))
             + _lrelu(readout(rs2, rm2, rc2))
             + _lrelu(readout(rs3, rm3, rc3)))
        ge_ref[...] = h
        h1 = _lrelu(_dot(h, wl1[...]) + bl1[...])
        h2 = _lrelu(_dot(h1, wl2[...]) + bl2[...])
        lg_ref[...] = _dot(h2, wl3[...]) + bl3[...]


_f32 = jnp.float32


def _tc_conv(part, cnt, t, kcol, Wl, Wr, brow, wp_pad):
    return pl.pallas_call(
        _conv_body,
        grid=(NBLK,),
        in_specs=[
            pl.BlockSpec((BR, 128), lambda i: (i, 0)),
            pl.BlockSpec((BR, 16), lambda i: (i, 0)),
            pl.BlockSpec((BR, 128), lambda i: (i, 0)),
            pl.BlockSpec((BR, 1), lambda i: (i, 0)),
            pl.BlockSpec((F, F), lambda i: (0, 0)),
            pl.BlockSpec((F, F), lambda i: (0, 0)),
            pl.BlockSpec((1, F), lambda i: (0, 0)),
            pl.BlockSpec((F, F), lambda i: (0, 0)),
        ],
        out_specs=[
            pl.BlockSpec((BR, F), lambda i: (i, 0)),
            pl.BlockSpec((BR, 16), lambda i: (i, 0)),
            pl.BlockSpec((BR, 1), lambda i: (i, 0)),
            pl.BlockSpec((BR, 1), lambda i: (i, 0)),
        ],
        out_shape=[
            jax.ShapeDtypeStruct((N_P, F), _f32),
            jax.ShapeDtypeStruct((N_P, 16), _f32),
            jax.ShapeDtypeStruct((N_P, 1), _f32),
            jax.ShapeDtypeStruct((N_P, 1), _f32),
        ],
    )(part, cnt, t, kcol, Wl, Wr, brow, wp_pad)


def _tc_pool(xo, epart, dinv, hcol, kcol, bcol, er, dr, hr, kR, bR, bp):
    blk1 = lambda: pl.BlockSpec((BR, 1), lambda i: (i, 0))
    row = lambda: pl.BlockSpec((JR, 128), lambda i: (0, 0))
    return pl.pallas_call(
        _pool_body,
        grid=(NBLK,),
        in_specs=[
            pl.BlockSpec((BR, F), lambda i: (i, 0)),
            pl.BlockSpec((BR, 16), lambda i: (i, 0)),
            blk1(), blk1(), blk1(), blk1(),
            row(), row(), row(), row(), row(),
            pl.BlockSpec((1, 1), lambda i: (0, 0)),
        ],
        out_specs=[
            pl.BlockSpec((BR, 128), lambda i: (i, 0)),
            pl.BlockSpec((BR, 1), lambda i: (i, 0)),
            pl.BlockSpec((NG, 128), lambda i: (0, 0)),
            pl.BlockSpec((NG, 128), lambda i: (0, 0)),
            pl.BlockSpec((NG, 128), lambda i: (0, 0)),
        ],
        out_shape=[
            jax.ShapeDtypeStruct((N_P, 128), _f32),
            jax.ShapeDtypeStruct((N_P, 1), _f32),
            jax.ShapeDtypeStruct((NG, 128), _f32),
            jax.ShapeDtypeStruct((NG, 128), _f32),
            jax.ShapeDtypeStruct((NG, 128), _f32),
        ],
    )(xo, epart, dinv, hcol, kcol, bcol, er, dr, hr, kR, bR, bp)


def _tc_ytab(degpart, t):
    return pl.pallas_call(
        _ytab_body,
        grid=(NBLK,),
        in_specs=[
            pl.BlockSpec((BR, 16), lambda i: (i, 0)),
            pl.BlockSpec((BR, 128), lambda i: (i, 0)),
        ],
        out_specs=[
            pl.BlockSpec((BR, 128), lambda i: (i, 0)),
            pl.BlockSpec((BR, 1), lambda i: (i, 0)),
        ],
        out_shape=[
            jax.ShapeDtypeStruct((N_P, 128), _f32),
            jax.ShapeDtypeStruct((N_P, 1), _f32),
        ],
    )(degpart, t)


def _tc_head(t, Spart, dinvS, kcol, reads, WL1, bL1, WL2, bL2, WL3p, bL3p):
    full = lambda shp: pl.BlockSpec(shp, lambda i: (0, 0))
    rspecs = []
    rargs = []
    for rs, rm, rc in reads:
        rspecs += [full((NG, 128))] * 3
        rargs += [rs, rm, rc]
    return pl.pallas_call(
        _head_body,
        grid=(NBLK,),
        in_specs=[
            pl.BlockSpec((BR, 128), lambda i: (i, 0)),
            pl.BlockSpec((BR, 128), lambda i: (i, 0)),
            pl.BlockSpec((BR, 1), lambda i: (i, 0)),
            pl.BlockSpec((BR, 1), lambda i: (i, 0)),
        ] + rspecs + [
            full((2 * F, F)), full((1, F)),
            full((F, 64)), full((1, 64)),
            full((64, 128)), full((1, 128)),
        ],
        out_specs=[
            full((1, 1)), full((NG, 2 * F)), full((NG, 128)),
        ],
        out_shape=[
            jax.ShapeDtypeStruct((1, 1), _f32),
            jax.ShapeDtypeStruct((NG, 2 * F), _f32),
            jax.ShapeDtypeStruct((NG, 128), _f32),
        ],
        scratch_shapes=[pltpu.SMEM((2,), _f32)],
    )(t, Spart, dinvS, kcol, *rargs, WL1, bL1, WL2, bL2, WL3p, bL3p)


def kernel(x, edge_index, batch, W1l, b1, W1r, W2l, b2, W2r, W3l, b3, W3r,
           Wp1, bp1, Wp2, bp2, Wp3, bp3, WL1, bL1, WL2, bL2, WL3, bL3):
    f32 = jnp.float32
    src = edge_index[0]
    dst = edge_index[1]
    srcp = jnp.pad(src, (0, EP - E)).reshape(EPR, 128)
    dstp = jnp.pad(dst, (0, EP - E), constant_values=DUMMY).reshape(EPR, 128)
    t = jnp.pad(x, ((0, PAD), (0, 0)))
    lane16 = jnp.arange(16)
    ones_t16 = (jnp.where(lane16[None, :] == 0, 1.0, 0.0)
                * jnp.ones((N_P, 1), f32))
    batchf = jnp.pad(batch.astype(f32), (0, PAD), constant_values=15.0)
    keep = jnp.pad(jnp.ones((N,), f32), (0, PAD))
    bcol = batchf.reshape(N_P, 1)
    bR = batchf.reshape(JR, 128)

    weights = [(W1l, b1, W1r, Wp1, bp1), (W2l, b2, W2r, Wp2, bp2),
               (W3l, b3, W3r, Wp3, bp3)]
    reads = []
    for (Wl, b, Wr, Wp, bp) in weights:
        part, cnt = _seg_call(TW_BIG, t, keep, srcp, dstp)
        kcol = keep.reshape(N_P, 1)
        xo, gt, hcol, dinv = _tc_conv(
            part, cnt, t, kcol, Wl, Wr, b.reshape(1, F),
            jnp.pad(Wp, ((0, 0), (0, F - 1))))
        epart = _seg_call(TW_SMALL, gt, keep, srcp, dstp)
        er = epart[:, 0].reshape(JR, 128)
        t, kn, rs, rm, rc = _tc_pool(
            xo, epart, dinv, hcol, kcol, bcol,
            er, dinv.reshape(JR, 128), hcol.reshape(JR, 128),
            keep.reshape(JR, 128), bR, bp.reshape(1, 1))
        keep = kn.reshape(N_P)
        reads.append((rs, rm, rc))

    degpart = _seg_call(TW_SMALL, ones_t16, keep, dstp, srcp)
    yt, dinvS = _tc_ytab(degpart, t)
    Spart, _cnt_unused = _seg_call(TW_BIG, yt, keep, srcp, dstp)
    mean1, ge, lgp = _tc_head(
        t, Spart, dinvS, keep.reshape(N_P, 1), reads,
        WL1, bL1.reshape(1, F), WL2, bL2.reshape(1, 64),
        jnp.pad(WL3, ((0, 0), (0, 128 - 30))),
        jnp.pad(bL3, (0, 128 - 30)).reshape(1, 128))
    return lgp[:NG, :30], mean1[0, 0], ge


# speedup vs baseline: 5.0819x; 1.8949x over previous
import functools

import jax
import jax.numpy as jnp
from jax import lax
from jax.experimental import pallas as pl
from jax.experimental.pallas import tpu as pltpu
from jax.experimental.pallas import tpu_sc as plsc

N = 10000
E = 320000
F = 128
NG = 16
N_P = 10112
PAD = N_P - N
DUMMY = N
NW = 32
CH = 80
EW = CH * 128
EP = EW * NW
EPR = EP // 128
BR = 632
NBLK = N_P // BR
JR = N_P // 128
TW_BIG = 128
TW_SMALL = 16
NHALF = N_P // 2
ACC_ROWS = 5120
DUMMY_L = NHALF
CAP = EP // 16 + 32


def _seg_body(TW, table_h, keep_h, a_h, b_h, *refs):
    with_cnt = TW == 128
    if with_cnt:
        (out_h, out16_h, keep_v, ab, bb, cpk, ia, ib, r0, r1, ones16,
         g0, g1, s0, s1, t0, t1, acc, acc16) = refs
    else:
        (out_h, keep_v, ab, bb, cpk, ia, ib, r0, r1,
         g0, g1, s0, s1, acc) = refs
    cid = lax.axis_index("c")
    sid = lax.axis_index("s")
    lo = cid * NHALF

    pltpu.sync_copy(keep_h, keep_v)

    @pl.loop(0, 128)
    def _z(r):
        for kk in range(TW // 16):
            r0[r, pl.ds(kk * 16, 16)] = jnp.zeros((16,), jnp.float32)

    zbase = sid * (ACC_ROWS // 16)
    for q in range(2):
        pltpu.sync_copy(r0, acc.at[pl.ds(zbase + q * 128, 128)])
    pltpu.sync_copy(r0.at[pl.ds(0, 64)], acc.at[pl.ds(zbase + 256, 64)])
    if with_cnt:
        @pl.loop(0, 128)
        def _z16(r):
            ones16[r, pl.ds(0, 16)] = jnp.zeros((16,), jnp.float32)

        for q in range(2):
            pltpu.sync_copy(ones16, acc16.at[pl.ds(zbase + q * 128, 128)])
        pltpu.sync_copy(ones16.at[pl.ds(0, 64)],
                        acc16.at[pl.ds(zbase + 256, 64)])
        col0 = (lax.broadcasted_iota(jnp.int32, (16,), 0) == 0
                ).astype(jnp.float32)

        @pl.loop(0, 128)
        def _o16(r):
            ones16[r, pl.ds(0, 16)] = col0

    plsc.subcore_barrier()

    row0 = sid * (EPR // 16)

    def _m(m, cnt):
        mrow = row0 + m * 16
        pltpu.sync_copy(a_h.at[pl.ds(mrow, 16)], ab)
        pltpu.sync_copy(b_h.at[pl.ds(mrow, 16)], bb)

        def _j(j, cnt):
            def _k(k, cnt):
                a16 = ab[j, pl.ds(k * 16, 16)]
                b16 = bb[j, pl.ds(k * 16, 16)]
                em = (plsc.load_gather(keep_v, [a16])
                      * plsc.load_gather(keep_v, [b16]))
                dl = b16 - lo
                live = (em > 0.0) & (dl >= 0) & (dl < NHALF)
                plsc.store_compressed(
                    cpk.at[pl.ds(cnt, 16)],
                    a16 | lax.shift_left(dl, 14), mask=live)
                return cnt + jnp.max(
                    plsc.all_reduce_population_count(live))

            return pl.loop(0, 8, init_carry=cnt)(_k)

        return pl.loop(0, 16, init_carry=cnt)(_j)

    cnt = pl.loop(0, EPR // 256, init_carry=jnp.int32(0))(_m)

    pad_to = ((cnt + 127) // 128) * 128

    @pl.loop(0, 8)
    def _t(p):
        off = cnt + p * 16

        @pl.when(off < pad_to)
        def _():
            cpk[pl.ds(off, 16)] = jnp.full((16,), DUMMY_L << 14, jnp.int32)

    npair = pad_to // 256
    rem = pad_to - npair * 256

    def _unpack(c0, qb):
        @pl.loop(0, 8)
        def _u(k):
            v = cpk[pl.ds(c0 + k * 16, 16)]
            ia[qb, pl.ds(k * 16, 16)] = v & 0x3FFF
            ib[qb, pl.ds(k * 16, 16)] = lax.shift_right_logical(v, 14)

    def _start(qb, rb, gs):
        pltpu.async_copy(table_h.at[ia.at[qb]], rb, gs)

    def _chunk(qb, rb, gs, ss, ts):
        pltpu.make_async_copy(table_h.at[ia.at[qb]], rb, gs).wait()
        pltpu.async_copy(rb, acc.at[ib.at[qb]], ss, add=True)
        if with_cnt:
            pltpu.async_copy(ones16, acc16.at[ib.at[qb]], ts, add=True)

    def _drain(qb, rb, ss, ts):
        pltpu.make_async_copy(rb, acc.at[ib.at[qb]], ss).wait()
        if with_cnt:
            pltpu.make_async_copy(ones16, acc16.at[ib.at[qb]], ts).wait()

    @pl.loop(0, npair)
    def _p(p):
        c0 = p * 256
        _unpack(c0, 0)
        _unpack(c0 + 128, 1)
        _start(0, r0, g0)
        _start(1, r1, g1)
        _chunk(0, r0, g0, s0, with_cnt and t0)
        _chunk(1, r1, g1, s1, with_cnt and t1)
        _drain(0, r0, s0, with_cnt and t0)
        _drain(1, r1, s1, with_cnt and t1)

    @pl.when(rem > 0)
    def _tail():
        _unpack(npair * 256, 0)
        _start(0, r0, g0)
        _chunk(0, r0, g0, s0, with_cnt and t0)
        _drain(0, r0, s0, with_cnt and t0)

    plsc.subcore_barrier()

    @pl.when(sid < 8)
    def _out():
        obase = sid * BR
        for q in range(4):
            pltpu.sync_copy(acc.at[pl.ds(obase + q * 128, 128)],
                            out_h.at[pl.ds(lo + obase + q * 128, 128)])
        pltpu.sync_copy(acc.at[pl.ds(obase + 512, 120)],
                        out_h.at[pl.ds(lo + obase + 512, 120)])
        if with_cnt:
            for q in range(4):
                pltpu.sync_copy(
                    acc16.at[pl.ds(obase + q * 128, 128)],
                    out16_h.at[pl.ds(lo + obase + q * 128, 128)])
            pltpu.sync_copy(acc16.at[pl.ds(obase + 512, 120)],
                            out16_h.at[pl.ds(lo + obase + 512, 120)])


def _make_seg(TW):
    mesh = plsc.VectorSubcoreMesh(core_axis_name="c", subcore_axis_name="s")
    with_cnt = TW == 128
    out_type = jax.ShapeDtypeStruct((N_P, TW), jnp.float32)
    if with_cnt:
        out_type = [out_type,
                    jax.ShapeDtypeStruct((N_P, 16), jnp.float32)]
    scratch = [
        pltpu.VMEM((N_P,), jnp.float32),
        pltpu.VMEM((16, 128), jnp.int32),
        pltpu.VMEM((16, 128), jnp.int32),
        pltpu.VMEM((CAP,), jnp.int32),
        pltpu.VMEM((2, 128), jnp.int32),
        pltpu.VMEM((2, 128), jnp.int32),
        pltpu.VMEM((128, TW), jnp.float32),
        pltpu.VMEM((128, TW), jnp.float32),
    ]
    if with_cnt:
        scratch += [pltpu.VMEM((128, 16), jnp.float32)]
        scratch += [pltpu.SemaphoreType.DMA] * 6
        scratch += [pltpu.VMEM_SHARED((ACC_ROWS, TW), jnp.float32),
                    pltpu.VMEM_SHARED((ACC_ROWS, 16), jnp.float32)]
    else:
        scratch += [pltpu.SemaphoreType.DMA] * 4
        scratch += [pltpu.VMEM_SHARED((ACC_ROWS, TW), jnp.float32)]
    return functools.partial(
        pl.kernel,
        out_type=out_type,
        mesh=mesh,
        scratch_types=scratch,
        compiler_params=pltpu.CompilerParams(needs_layout_passes=False,
                                             use_tc_tiling_on_sc=False),
    )(functools.partial(_seg_body, TW))


def _seg_call(TW, table, keep, a, b):
    return _make_seg(TW)(table, keep, a, b)


def _lrelu(v):
    return jnp.where(v >= 0, v, 0.1 * v)


def _dot(a, b):
    return lax.dot_general(a, b, (((1,), (0,)), ((), ())),
                           preferred_element_type=jnp.float32)


def _conv_body(s_ref, cnt_ref, t_ref, k_ref, wl_ref, wr_ref, b_ref, wp_ref,
               xo_ref, gt_ref, h_ref, dinv_ref):
    srows = s_ref[...]
    c = cnt_ref[...][:, 0:1]
    xin = t_ref[...]
    mean = srows / jnp.maximum(c, 1.0)
    z = _dot(mean, wl_ref[...]) + _dot(xin, wr_ref[...]) + b_ref[...]
    xo = _lrelu(z)
    xo_ref[...] = xo
    hc = _dot(xo, wp_ref[...])[:, 0:1]
    kf = k_ref[...]
    deg = c + kf
    dinv = jnp.where(deg > 0, lax.rsqrt(jnp.maximum(deg, 1e-30)), 0.0)
    lane16 = lax.broadcasted_iota(jnp.int32, (BR, 16), 1)
    gt_ref[...] = jnp.where(lane16 == 0, dinv * hc, 0.0)
    h_ref[...] = hc
    dinv_ref[...] = dinv


def _pool_body(xo_ref, e_ref, dinv_ref, h_ref, kc_ref, bc_ref,
               er_ref, dr_ref, hr_ref, kR_ref, bR_ref, bp_ref,
               xn_ref, kn_ref, rs_ref, rm_ref, rc_ref):
    i = pl.program_id(0)
    bp = bp_ref[0, 0]
    dinv = dinv_ref[...]
    sc = (dinv * e_ref[...][:, 0:1]
          + dinv * dinv * kc_ref[...] * h_ref[...] + bp)
    kc = kc_ref[...]
    bc = bc_ref[...]
    kR = kR_ref[...]
    bR = bR_ref[...]
    kp_col = jnp.zeros((BR, 1), jnp.float32)
    for g in range(NG):
        ckg = jnp.sum(jnp.where(bR == float(g), kR, 0.0))
        kpg = jnp.floor((ckg + 1.0) * 0.5)
        kp_col = kp_col + jnp.where(bc == float(g), kpg, 0.0)
    ii = i * BR + lax.broadcasted_iota(jnp.int32, (BR, 128), 0)

    def jstep(jc, acc):
        dj = dr_ref[jc].reshape(1, 128)
        kj = kR_ref[jc].reshape(1, 128)
        sj = (dj * er_ref[jc].reshape(1, 128)
              + dj * dj * kj * hr_ref[jc].reshape(1, 128) + bp)
        bj = bR_ref[jc].reshape(1, 128)
        jj = jc * 128 + lax.broadcasted_iota(jnp.int32, (BR, 128), 1)
        beat = (sj > sc) | ((sj == sc) & (jj < ii))
        m = (bj == bc) & (kj > 0.0)
        return acc + jnp.sum(jnp.where(beat & m, 1.0, 0.0), axis=1,
                             keepdims=True)

    bmin = jnp.min(bc)
    bmax = jnp.max(bc)
    first = jnp.sum(jnp.where(bR < bmin, 1.0, 0.0))
    last = jnp.sum(jnp.where(bR <= bmax, 1.0, 0.0)) - 1.0
    jlo = (first // 128.0).astype(jnp.int32)
    jhi = (last // 128.0).astype(jnp.int32) + 1
    rank = lax.fori_loop(jlo, jhi, jstep, jnp.zeros((BR, 1), jnp.float32))
    kn = jnp.where((kc > 0.0) & (rank < kp_col), 1.0, 0.0)
    x_new = jnp.where(kn > 0.0, xo_ref[...] * jnp.tanh(sc), 0.0)
    xn_ref[...] = x_new
    kn_ref[...] = kn

    @pl.when(i == 0)
    def _init():
        rs_ref[...] = jnp.zeros((NG, 128), jnp.float32)
        rm_ref[...] = jnp.full((NG, 128), -3.4e38, jnp.float32)
        rc_ref[...] = jnp.zeros((NG, 128), jnp.float32)

    for g in range(NG):
        selg = bc == float(g)
        rs_ref[g:g + 1, :] += jnp.sum(jnp.where(selg, x_new, 0.0), axis=0,
                                      keepdims=True)
        rm_ref[g:g + 1, :] = jnp.maximum(
            rm_ref[g:g + 1, :],
            jnp.max(jnp.where(selg & (kn > 0.0), x_new, -3.4e38), axis=0,
                    keepdims=True))
        rc_ref[g:g + 1, :] += jnp.sum(jnp.where(selg, kn, 0.0), axis=0,
                                      keepdims=True)


def _ytab_body(d_ref, t_ref, y_ref, dinv_ref):
    degS = d_ref[...][:, 0:1]
    dinvS = jnp.where(degS > 0, lax.rsqrt(jnp.maximum(degS, 1e-30)), 0.0)
    y_ref[...] = dinvS * t_ref[...]
    dinv_ref[...] = dinvS


def _head_body(t_ref, S_ref, dinvS_ref, k_ref,
               rs1, rm1, rc1, rs2, rm2, rc2, rs3, rm3, rc3,
               wl1, bl1, wl2, bl2, wl3, bl3,
               mean_ref, ge_ref, lg_ref, acc_ref):
    i = pl.program_id(0)

    @pl.when(i == 0)
    def _init():
        acc_ref[0] = 0.0
        acc_ref[1] = 0.0

    S = S_ref[...]
    agg = dinvS_ref[...] * S
    info = t_ref[...] - agg
    sn = jnp.sum(jnp.abs(info), axis=1, keepdims=True)
    kf = k_ref[...]
    acc_ref[0] += jnp.sum(sn * kf)
    acc_ref[1] += jnp.sum(kf)

    @pl.when(i == NBLK - 1)
    def _final():
        mean_ref[...] = (acc_ref[0] / acc_ref[1])[None, None]

        def readout(rs, rm, rc):
            mn = rs[...] / jnp.maximum(rc[...], 1.0)
            mx = jnp.where(rm[...] > -1e37, rm[...], 0.0)
            return jnp.concatenate([mx, mn], axis=1)

        h = (_lrelu(readout(rs1, rm1, rc1))
             + _lrelu(readout(rs2, rm2, rc2))
             + _lrelu(readout(rs3, rm3, rc3)))
        ge_ref[...] = h
        h1 = _lrelu(_dot(h, wl1[...]) + bl1[...])
        h2 = _lrelu(_dot(h1, wl2[...]) + bl2[...])
        lg_ref[...] = _dot(h2, wl3[...]) + bl3[...]


_f32 = jnp.float32


def _tc_conv(part, cnt, t, kcol, Wl, Wr, brow, wp_pad):
    return pl.pallas_call(
        _conv_body,
        grid=(NBLK,),
        in_specs=[
            pl.BlockSpec((BR, 128), lambda i: (i, 0)),
            pl.BlockSpec((BR, 16), lambda i: (i, 0)),
            pl.BlockSpec((BR, 128), lambda i: (i, 0)),
            pl.BlockSpec((BR, 1), lambda i: (i, 0)),
            pl.BlockSpec((F, F), lambda i: (0, 0)),
            pl.BlockSpec((F, F), lambda i: (0, 0)),
            pl.BlockSpec((1, F), lambda i: (0, 0)),
            pl.BlockSpec((F, F), lambda i: (0, 0)),
        ],
        out_specs=[
            pl.BlockSpec((BR, F), lambda i: (i, 0)),
            pl.BlockSpec((BR, 16), lambda i: (i, 0)),
            pl.BlockSpec((BR, 1), lambda i: (i, 0)),
            pl.BlockSpec((BR, 1), lambda i: (i, 0)),
        ],
        out_shape=[
            jax.ShapeDtypeStruct((N_P, F), _f32),
            jax.ShapeDtypeStruct((N_P, 16), _f32),
            jax.ShapeDtypeStruct((N_P, 1), _f32),
            jax.ShapeDtypeStruct((N_P, 1), _f32),
        ],
    )(part, cnt, t, kcol, Wl, Wr, brow, wp_pad)


def _tc_pool(xo, epart, dinv, hcol, kcol, bcol, er, dr, hr, kR, bR, bp):
    blk1 = lambda: pl.BlockSpec((BR, 1), lambda i: (i, 0))
    row = lambda: pl.BlockSpec((JR, 128), lambda i: (0, 0))
    return pl.pallas_call(
        _pool_body,
        grid=(NBLK,),
        in_specs=[
            pl.BlockSpec((BR, F), lambda i: (i, 0)),
            pl.BlockSpec((BR, 16), lambda i: (i, 0)),
            blk1(), blk1(), blk1(), blk1(),
            row(), row(), row(), row(), row(),
            pl.BlockSpec((1, 1), lambda i: (0, 0)),
        ],
        out_specs=[
            pl.BlockSpec((BR, 128), lambda i: (i, 0)),
            pl.BlockSpec((BR, 1), lambda i: (i, 0)),
            pl.BlockSpec((NG, 128), lambda i: (0, 0)),
            pl.BlockSpec((NG, 128), lambda i: (0, 0)),
            pl.BlockSpec((NG, 128), lambda i: (0, 0)),
        ],
        out_shape=[
            jax.ShapeDtypeStruct((N_P, 128), _f32),
            jax.ShapeDtypeStruct((N_P, 1), _f32),
            jax.ShapeDtypeStruct((NG, 128), _f32),
            jax.ShapeDtypeStruct((NG, 128), _f32),
            jax.ShapeDtypeStruct((NG, 128), _f32),
        ],
    )(xo, epart, dinv, hcol, kcol, bcol, er, dr, hr, kR, bR, bp)


def _tc_ytab(degpart, t):
    return pl.pallas_call(
        _ytab_body,
        grid=(NBLK,),
        in_specs=[
            pl.BlockSpec((BR, 16), lambda i: (i, 0)),
            pl.BlockSpec((BR, 128), lambda i: (i, 0)),
        ],
        out_specs=[
            pl.BlockSpec((BR, 128), lambda i: (i, 0)),
            pl.BlockSpec((BR, 1), lambda i: (i, 0)),
        ],
        out_shape=[
            jax.ShapeDtypeStruct((N_P, 128), _f32),
            jax.ShapeDtypeStruct((N_P, 1), _f32),
        ],
    )(degpart, t)


def _tc_head(t, Spart, dinvS, kcol, reads, WL1, bL1, WL2, bL2, WL3p, bL3p):
    full = lambda shp: pl.BlockSpec(shp, lambda i: (0, 0))
    rspecs = []
    rargs = []
    for rs, rm, rc in reads:
        rspecs += [full((NG, 128))] * 3
        rargs += [rs, rm, rc]
    return pl.pallas_call(
        _head_body,
        grid=(NBLK,),
        in_specs=[
            pl.BlockSpec((BR, 128), lambda i: (i, 0)),
            pl.BlockSpec((BR, 128), lambda i: (i, 0)),
            pl.BlockSpec((BR, 1), lambda i: (i, 0)),
            pl.BlockSpec((BR, 1), lambda i: (i, 0)),
        ] + rspecs + [
            full((2 * F, F)), full((1, F)),
            full((F, 64)), full((1, 64)),
            full((64, 128)), full((1, 128)),
        ],
        out_specs=[
            full((1, 1)), full((NG, 2 * F)), full((NG, 128)),
        ],
        out_shape=[
            jax.ShapeDtypeStruct((1, 1), _f32),
            jax.ShapeDtypeStruct((NG, 2 * F), _f32),
            jax.ShapeDtypeStruct((NG, 128), _f32),
        ],
        scratch_shapes=[pltpu.SMEM((2,), _f32)],
    )(t, Spart, dinvS, kcol, *rargs, WL1, bL1, WL2, bL2, WL3p, bL3p)


def kernel(x, edge_index, batch, W1l, b1, W1r, W2l, b2, W2r, W3l, b3, W3r,
           Wp1, bp1, Wp2, bp2, Wp3, bp3, WL1, bL1, WL2, bL2, WL3, bL3):
    f32 = jnp.float32
    src = edge_index[0]
    dst = edge_index[1]
    srcp = jnp.pad(src, (0, EP - E)).reshape(EPR, 128)
    dstp = jnp.pad(dst, (0, EP - E), constant_values=DUMMY).reshape(EPR, 128)
    t = jnp.pad(x, ((0, PAD), (0, 0)))
    lane16 = jnp.arange(16)
    ones_t16 = (jnp.where(lane16[None, :] == 0, 1.0, 0.0)
                * jnp.ones((N_P, 1), f32))
    batchf = jnp.pad(batch.astype(f32), (0, PAD), constant_values=15.0)
    keep = jnp.pad(jnp.ones((N,), f32), (0, PAD))
    bcol = batchf.reshape(N_P, 1)
    bR = batchf.reshape(JR, 128)

    weights = [(W1l, b1, W1r, Wp1, bp1), (W2l, b2, W2r, Wp2, bp2),
               (W3l, b3, W3r, Wp3, bp3)]
    reads = []
    for (Wl, b, Wr, Wp, bp) in weights:
        part, cnt = _seg_call(TW_BIG, t, keep, srcp, dstp)
        kcol = keep.reshape(N_P, 1)
        xo, gt, hcol, dinv = _tc_conv(
            part, cnt, t, kcol, Wl, Wr, b.reshape(1, F),
            jnp.pad(Wp, ((0, 0), (0, F - 1))))
        epart = _seg_call(TW_SMALL, gt, keep, srcp, dstp)
        er = epart[:, 0].reshape(JR, 128)
        t, kn, rs, rm, rc = _tc_pool(
            xo, epart, dinv, hcol, kcol, bcol,
            er, dinv.reshape(JR, 128), hcol.reshape(JR, 128),
            keep.reshape(JR, 128), bR, bp.reshape(1, 1))
        keep = kn.reshape(N_P)
        reads.append((rs, rm, rc))

    degpart = _seg_call(TW_SMALL, ones_t16, keep, dstp, srcp)
    yt, dinvS = _tc_ytab(degpart, t)
    Spart, _cnt_unused = _seg_call(TW_BIG, yt, keep, srcp, dstp)
    mean1, ge, lgp = _tc_head(
        t, Spart, dinvS, keep.reshape(N_P, 1), reads,
        WL1, bL1.reshape(1, F), WL2, bL2.reshape(1, 64),
        jnp.pad(WL3, ((0, 0), (0, 128 - 30))),
        jnp.pad(bL3, (0, 128 - 30)).reshape(1, 128))
    return lgp[:NG, :30], mean1[0, 0], ge
